# Initial kernel scaffold; baseline (speedup 1.0000x reference)
import jax
import jax.numpy as jnp
from jax.experimental import pallas as pl


def kernel(x, W):
    raise NotImplementedError("write your pallas kernel here")


# speedup vs baseline: 1.2347x; 1.2347x over previous
import math

import jax
import jax.numpy as jnp
import numpy as np
from jax import lax
from jax.experimental import pallas as pl
from jax.experimental.pallas import tpu as pltpu
from jax.experimental.pallas import tpu_sc as plsc

BLOCK_SIZE = 200
DMODEL = 32
BATCH = 4096
SCALE = math.sqrt(DMODEL)

NUM_CORES = 2
NUM_SUBCORES = 16
NUM_WORKERS = NUM_CORES * NUM_SUBCORES
SEQ_PER_WORKER = BATCH // NUM_WORKERS


def _positional_embeddings_scaled():
    pe = np.zeros((BLOCK_SIZE, DMODEL), dtype=np.float32)
    pos = np.arange(BLOCK_SIZE, dtype=np.float32)[:, None]
    i = np.arange(DMODEL // 2, dtype=np.float32)[None, :]
    denom = np.power(10000.0, 2.0 * i / DMODEL)
    pe[:, 0::2] = np.sin(pos / denom)
    pe[:, 1::2] = np.cos(pos / denom)
    return jnp.asarray(pe * SCALE)


_MESH = plsc.VectorSubcoreMesh(
    core_axis_name="c", subcore_axis_name="s",
    num_cores=NUM_CORES, num_subcores=NUM_SUBCORES,
)


@jax.jit
def _embed(x, w, pe_s):
    @pl.kernel(
        out_type=jax.ShapeDtypeStruct((BATCH, BLOCK_SIZE, DMODEL), jnp.float32),
        mesh=_MESH,
        scratch_types=[
            pltpu.VMEM((BLOCK_SIZE,), jnp.int32),
            pltpu.VMEM((BLOCK_SIZE, DMODEL), jnp.float32),
            pltpu.VMEM((BLOCK_SIZE, DMODEL), jnp.float32),
            pltpu.SemaphoreType.DMA,
        ],
        compiler_params=pltpu.CompilerParams(use_tc_tiling_on_sc=False),
    )
    def body(x_hbm, w_hbm, pe_hbm, out_hbm, idx_v, rows_v, pe_v, sem):
        wid = lax.axis_index("s") * NUM_CORES + lax.axis_index("c")
        pltpu.sync_copy(pe_hbm, pe_v)

        def seq_body(k, carry):
            b = wid * SEQ_PER_WORKER + k
            pltpu.sync_copy(x_hbm.at[b], idx_v)
            pltpu.async_copy(w_hbm.at[idx_v], rows_v, sem).wait()

            def row_body(r, carry2):
                base = r * 8
                for rr in range(8):
                    row = base + rr
                    for c in range(2):
                        sl = pl.ds(c * 16, 16)
                        rows_v[row, sl] = rows_v[row, sl] * SCALE + pe_v[row, sl]
                return carry2

            lax.fori_loop(0, BLOCK_SIZE // 8, row_body, 0, unroll=False)
            pltpu.sync_copy(rows_v, out_hbm.at[b])
            return carry

        lax.fori_loop(0, SEQ_PER_WORKER, seq_body, 0, unroll=False)

    return body(x, w, pe_s)


def kernel(x, W):
    pe_s = _positional_embeddings_scaled()
    return _embed(x.astype(jnp.int32), W, pe_s)


# speedup vs baseline: 1.4924x; 1.2088x over previous
import math

import jax
import jax.numpy as jnp
import numpy as np
from jax import lax
from jax.experimental import pallas as pl
from jax.experimental.pallas import tpu as pltpu
from jax.experimental.pallas import tpu_sc as plsc

BLOCK_SIZE = 200
DMODEL = 32
BATCH = 4096
SCALE = math.sqrt(DMODEL)

NUM_CORES = 2
NUM_SUBCORES = 16
NUM_WORKERS = NUM_CORES * NUM_SUBCORES
SEQ_PER_WORKER = BATCH // NUM_WORKERS
IDX_PER_WORKER = SEQ_PER_WORKER * BLOCK_SIZE
NBUF = 4
NGROUPS = SEQ_PER_WORKER // NBUF
CHUNK_BYTES = BLOCK_SIZE * DMODEL * 4


def _positional_embeddings_scaled():
    pe = np.zeros((BLOCK_SIZE, DMODEL), dtype=np.float32)
    pos = np.arange(BLOCK_SIZE, dtype=np.float32)[:, None]
    i = np.arange(DMODEL // 2, dtype=np.float32)[None, :]
    denom = np.power(10000.0, 2.0 * i / DMODEL)
    pe[:, 0::2] = np.sin(pos / denom)
    pe[:, 1::2] = np.cos(pos / denom)
    return jnp.asarray(pe * SCALE)


_MESH = plsc.VectorSubcoreMesh(
    core_axis_name="c", subcore_axis_name="s",
    num_cores=NUM_CORES, num_subcores=NUM_SUBCORES,
)

_CHUNK = (BLOCK_SIZE, DMODEL)


@jax.jit
def _embed(x_flat, w, pe_s):
    @pl.kernel(
        out_type=jax.ShapeDtypeStruct((BATCH * BLOCK_SIZE, DMODEL), jnp.float32),
        mesh=_MESH,
        scratch_types=(
            [pltpu.VMEM((IDX_PER_WORKER,), jnp.int32)]
            + [pltpu.VMEM(_CHUNK, jnp.float32) for _ in range(2 * NBUF + 1)]
            + [pltpu.SemaphoreType.DMA for _ in range(2 * NBUF)]
        ),
        compiler_params=pltpu.CompilerParams(use_tc_tiling_on_sc=False),
    )
    def body(x_hbm, w_hbm, pe_hbm, out_hbm, idx_all, *rest):
        gbufs = rest[:NBUF]
        wbufs = rest[NBUF:2 * NBUF]
        pe_v = rest[2 * NBUF]
        gsems = rest[2 * NBUF + 1:2 * NBUF + 1 + NBUF]
        wsems = rest[2 * NBUF + 1 + NBUF:]

        wid = lax.axis_index("s") * NUM_CORES + lax.axis_index("c")
        base = wid * IDX_PER_WORKER
        pltpu.sync_copy(pe_hbm, pe_v)
        pltpu.sync_copy(x_hbm.at[pl.ds(base, IDX_PER_WORKER)], idx_all)

        def gather(c, s):
            pltpu.async_copy(
                w_hbm.at[idx_all.at[pl.ds(c * BLOCK_SIZE, BLOCK_SIZE)]],
                gbufs[s], gsems[s])

        for s in range(NBUF):
            gather(s, s)

        def group(g, carry):
            for s in range(NBUF):
                c = g * NBUF + s
                pltpu.make_async_copy(
                    w_hbm.at[pl.ds(0, BLOCK_SIZE)], gbufs[s], gsems[s]).wait()
                @pl.when(g > 0)
                def _():
                    pltpu.make_async_copy(
                        wbufs[s], out_hbm.at[pl.ds(0, BLOCK_SIZE)],
                        wsems[s]).wait()

                def row_body(r, cy):
                    b8 = r * 8
                    for rr in range(8):
                        row = b8 + rr
                        for h in range(2):
                            sl = pl.ds(h * 16, 16)
                            wbufs[s][row, sl] = (
                                gbufs[s][row, sl] * SCALE + pe_v[row, sl])
                    return cy

                lax.fori_loop(0, BLOCK_SIZE // 8, row_body, 0)

                pltpu.async_copy(
                    wbufs[s],
                    out_hbm.at[pl.ds(base + c * BLOCK_SIZE, BLOCK_SIZE)],
                    wsems[s])

                @pl.when(c + NBUF < SEQ_PER_WORKER)
                def _():
                    gather(c + NBUF, s)
            return carry

        lax.fori_loop(0, NGROUPS, group, 0)

        for s in range(NBUF):
            pltpu.make_async_copy(
                wbufs[s], out_hbm.at[pl.ds(0, BLOCK_SIZE)], wsems[s]).wait()

    return body(x_flat, w, pe_s)


def kernel(x, W):
    pe_s = _positional_embeddings_scaled()
    out = _embed(x.reshape(-1).astype(jnp.int32), W, pe_s)
    return out.reshape(BATCH, BLOCK_SIZE, DMODEL)


# speedup vs baseline: 1.4935x; 1.0008x over previous
import math

import jax
import jax.numpy as jnp
import numpy as np
from jax import lax
from jax.experimental import pallas as pl
from jax.experimental.pallas import tpu as pltpu
from jax.experimental.pallas import tpu_sc as plsc

BLOCK_SIZE = 200
DMODEL = 32
BATCH = 4096
SCALE = math.sqrt(DMODEL)

NUM_CORES = 2
NUM_SUBCORES = 16
NUM_WORKERS = NUM_CORES * NUM_SUBCORES
SEQ_PER_WORKER = BATCH // NUM_WORKERS
NBUF = 4
NGROUPS = SEQ_PER_WORKER // NBUF


def _positional_embeddings_scaled():
    pe = np.zeros((BLOCK_SIZE, DMODEL), dtype=np.float32)
    pos = np.arange(BLOCK_SIZE, dtype=np.float32)[:, None]
    i = np.arange(DMODEL // 2, dtype=np.float32)[None, :]
    denom = np.power(10000.0, 2.0 * i / DMODEL)
    pe[:, 0::2] = np.sin(pos / denom)
    pe[:, 1::2] = np.cos(pos / denom)
    return jnp.asarray(pe * SCALE)


_MESH = plsc.VectorSubcoreMesh(
    core_axis_name="c", subcore_axis_name="s",
    num_cores=NUM_CORES, num_subcores=NUM_SUBCORES,
)

_CHUNK = (BLOCK_SIZE, DMODEL)


@jax.jit
def _embed(x, w, pe_s):
    @pl.kernel(
        out_type=jax.ShapeDtypeStruct((BATCH, BLOCK_SIZE, DMODEL), jnp.float32),
        mesh=_MESH,
        scratch_types=(
            [pltpu.VMEM((SEQ_PER_WORKER, BLOCK_SIZE), jnp.int32)]
            + [pltpu.VMEM(_CHUNK, jnp.float32) for _ in range(2 * NBUF + 1)]
            + [pltpu.SemaphoreType.DMA for _ in range(2 * NBUF)]
        ),
        compiler_params=pltpu.CompilerParams(use_tc_tiling_on_sc=False),
    )
    def body(x_hbm, w_hbm, pe_hbm, out_hbm, idx_all, *rest):
        gbufs = rest[:NBUF]
        wbufs = rest[NBUF:2 * NBUF]
        pe_v = rest[2 * NBUF]
        gsems = rest[2 * NBUF + 1:2 * NBUF + 1 + NBUF]
        wsems = rest[2 * NBUF + 1 + NBUF:]

        wid = lax.axis_index("s") * NUM_CORES + lax.axis_index("c")
        seq0 = wid * SEQ_PER_WORKER
        pltpu.sync_copy(pe_hbm, pe_v)
        pltpu.sync_copy(x_hbm.at[pl.ds(seq0, SEQ_PER_WORKER)], idx_all)

        def gather(c, s):
            pltpu.async_copy(w_hbm.at[idx_all.at[c]], gbufs[s], gsems[s])

        for s in range(NBUF):
            gather(s, s)

        def group(g, carry):
            for s in range(NBUF):
                c = g * NBUF + s
                pltpu.make_async_copy(
                    w_hbm.at[pl.ds(0, BLOCK_SIZE)], gbufs[s], gsems[s]).wait()
                @pl.when(g > 0)
                def _():
                    pltpu.make_async_copy(
                        wbufs[s], out_hbm.at[0], wsems[s]).wait()

                def row_body(r, cy):
                    b8 = r * 8
                    for rr in range(8):
                        row = b8 + rr
                        for h in range(2):
                            sl = pl.ds(h * 16, 16)
                            wbufs[s][row, sl] = (
                                gbufs[s][row, sl] * SCALE + pe_v[row, sl])
                    return cy

                lax.fori_loop(0, BLOCK_SIZE // 8, row_body, 0)

                pltpu.async_copy(wbufs[s], out_hbm.at[seq0 + c], wsems[s])

                @pl.when(c + NBUF < SEQ_PER_WORKER)
                def _():
                    gather(c + NBUF, s)
            return carry

        lax.fori_loop(0, NGROUPS, group, 0)

        for s in range(NBUF):
            pltpu.make_async_copy(
                wbufs[s], out_hbm.at[0], wsems[s]).wait()

    return body(x, w, pe_s)


def kernel(x, W):
    pe_s = _positional_embeddings_scaled()
    return _embed(x.astype(jnp.int32), W, pe_s)
